# baseline (speedup 1.0000x reference)
import jax
import jax.numpy as jnp
from jax import lax
from jax.experimental import pallas as pl
from jax.experimental.pallas import tpu as pltpu
from jax.experimental.pallas import tpu_sc as plsc

TOTAL_LEN = 1000000
EMBED_DIM = 64
OUT_DIM = 10
BATCH = 16384

NC = 2
NS = 16
NW = NC * NS
B_PER_W = BATCH // NW
GB = 4
LANES = 16


def _sc_body(xid_hbm, table_hbm, wsplat_hbm, bsplat_hbm, out_hbm,
             idx_v, rows_v, w_v, b_v, out_v, sem):
    wid = lax.axis_index("s") * NC + lax.axis_index("c")
    base = wid * B_PER_W

    pltpu.sync_copy(xid_hbm.at[pl.ds(base, B_PER_W)], idx_v)
    pltpu.sync_copy(wsplat_hbm, w_v)
    pltpu.sync_copy(bsplat_hbm, b_v)

    pltpu.async_copy(table_hbm.at[idx_v], rows_v, sem).wait()

    iota = lax.iota(jnp.int32, LANES)
    one = jnp.full((LANES,), 1, dtype=jnp.int32)

    def group_block(gb, carry):
        b0 = gb * (LANES * GB)
        bvecs = [b0 + g * LANES + iota for g in range(GB)]
        accs = [[b_v[pl.ds(o * LANES, LANES)] for o in range(OUT_DIM)]
                for g in range(GB)]
        dvec = jnp.zeros((LANES,), dtype=jnp.int32)
        for d in range(EMBED_DIM):
            hs = [plsc.load_gather(rows_v, [bvecs[g], dvec])
                  for g in range(GB)]
            for o in range(OUT_DIM):
                w = w_v[pl.ds((d * OUT_DIM + o) * LANES, LANES)]
                for g in range(GB):
                    accs[g][o] = accs[g][o] + hs[g] * w
            dvec = dvec + one
        for g in range(GB):
            obase = bvecs[g] * OUT_DIM
            for o in range(OUT_DIM):
                plsc.store_scatter(out_v, [obase + o], accs[g][o])
        return carry

    lax.fori_loop(0, B_PER_W // (LANES * GB), group_block, 0)

    pltpu.sync_copy(out_v, out_hbm.at[pl.ds(base * OUT_DIM, B_PER_W * OUT_DIM)])


@jax.jit
def _mlp3_sc(x_id, table, wsplat, bsplat):
    mesh = plsc.VectorSubcoreMesh(core_axis_name="c", subcore_axis_name="s")
    out_flat = pl.kernel(
        _sc_body,
        out_type=jax.ShapeDtypeStruct((BATCH * OUT_DIM,), jnp.float32),
        mesh=mesh,
        compiler_params=pltpu.CompilerParams(
            needs_layout_passes=False, use_tc_tiling_on_sc=False),
        scratch_types=[
            pltpu.VMEM((B_PER_W,), jnp.int32),
            pltpu.VMEM((B_PER_W, EMBED_DIM), jnp.float32),
            pltpu.VMEM((EMBED_DIM * OUT_DIM * LANES,), jnp.float32),
            pltpu.VMEM((OUT_DIM * LANES,), jnp.float32),
            pltpu.VMEM((B_PER_W * OUT_DIM,), jnp.float32),
            pltpu.SemaphoreType.DMA,
        ],
    )(x_id, table, wsplat, bsplat)
    return out_flat.reshape((BATCH, OUT_DIM))


def kernel(x_id, table, W, b):
    wsplat = jnp.broadcast_to(W.T[:, :, None], (EMBED_DIM, OUT_DIM, LANES))
    bsplat = jnp.broadcast_to(b[:, None], (OUT_DIM, LANES))
    return _mlp3_sc(x_id.astype(jnp.int32), table,
                    wsplat.reshape(-1), bsplat.reshape(-1))
